# Initial kernel scaffold; baseline (speedup 1.0000x reference)
import jax
import jax.numpy as jnp
from jax.experimental import pallas as pl


def kernel(x, edge_index, ptr, W, b):
    raise NotImplementedError("write your pallas kernel here")


# speedup vs baseline: 49.7305x; 49.7305x over previous
import functools

import jax
import jax.numpy as jnp
from jax import lax
from jax.experimental import pallas as pl
from jax.experimental.pallas import tpu as pltpu
from jax.experimental.pallas import tpu_sc as plsc

N = 10000
E = 320000
D_IN = 128
D_HID = 32

NC = 2
NS = 16
NW = NC * NS
EPT = E // NW
CH = 125
NCH = EPT // CH
NPAD = 10240
RPT = NPAD // NS

_mesh = plsc.VectorSubcoreMesh(core_axis_name="c", subcore_axis_name="s",
                               num_cores=NC, num_subcores=NS)
_sc_params = pltpu.CompilerParams(use_tc_tiling_on_sc=False)


@functools.partial(
    pl.kernel,
    out_type=jax.ShapeDtypeStruct((NC, NPAD), jnp.float32),
    mesh=_mesh,
    compiler_params=_sc_params,
    scratch_types=[
        pltpu.VMEM((NCH, CH), jnp.int32),
        pltpu.VMEM((128,), jnp.float32),
        pltpu.VMEM_SHARED((NPAD,), jnp.float32),
    ],
)
def _sc_degree(dst_hbm, zeros_hbm, deg_out, dst_v, ones_v, deg_sh):
    cid = lax.axis_index("c")
    sid = lax.axis_index("s")
    wid = cid * NS + sid
    pltpu.sync_copy(dst_hbm.at[wid], dst_v)
    for i in range(8):
        ones_v[pl.ds(i * 16, 16)] = jnp.full((16,), 1.0, jnp.float32)
    pltpu.sync_copy(zeros_hbm.at[pl.ds(sid * RPT, RPT)],
                    deg_sh.at[pl.ds(sid * RPT, RPT)])
    plsc.subcore_barrier()

    def body(j, carry):
        pltpu.sync_copy(ones_v.at[pl.ds(0, CH)], deg_sh.at[dst_v.at[j]],
                        add=True)
        return carry

    lax.fori_loop(0, NCH, body, 0)
    plsc.subcore_barrier()
    pltpu.sync_copy(deg_sh.at[pl.ds(sid * RPT, RPT)],
                    deg_out.at[cid, pl.ds(sid * RPT, RPT)])


def _tc_prescale_body(x_ref, w_ref, d0_ref, d1_ref, g_ref, dis_ref):
    deg = d0_ref[...] + d1_ref[...] + 1.0
    dis = lax.rsqrt(deg)
    h = jnp.dot(x_ref[...], w_ref[...], preferred_element_type=jnp.float32)
    g_ref[...] = h * dis
    dis_ref[...] = dis


@functools.partial(
    pl.kernel,
    out_type=jax.ShapeDtypeStruct((NC, NPAD, D_HID), jnp.float32),
    mesh=_mesh,
    compiler_params=_sc_params,
    scratch_types=[
        pltpu.VMEM((NCH, CH), jnp.int32),
        pltpu.VMEM((NCH, CH), jnp.int32),
        pltpu.VMEM((CH, D_HID), jnp.float32),
        pltpu.VMEM((CH, D_HID), jnp.float32),
        pltpu.VMEM((128, D_HID), jnp.float32),
        pltpu.VMEM_SHARED((NPAD, D_HID), jnp.float32),
        pltpu.SemaphoreType.DMA,
        pltpu.SemaphoreType.DMA,
    ],
)
def _sc_messages(src_hbm, dst_hbm, g_hbm, zeros_hbm, acc_out,
                 src_v, dst_v, msg0, msg1, zb, acc_sh, sem0, sem1):
    cid = lax.axis_index("c")
    sid = lax.axis_index("s")
    wid = cid * NS + sid
    pltpu.sync_copy(src_hbm.at[wid], src_v)
    pltpu.sync_copy(dst_hbm.at[wid], dst_v)
    pltpu.sync_copy(zeros_hbm, zb)
    for jj in range(RPT // 128):
        pltpu.sync_copy(zb, acc_sh.at[pl.ds(sid * RPT + jj * 128, 128)])
    plsc.subcore_barrier()

    pltpu.async_copy(g_hbm.at[src_v.at[0]], msg0, sem0)
    pltpu.async_copy(g_hbm.at[src_v.at[1]], msg1, sem1)

    def body(j2, carry):
        j = j2 * 2
        pltpu.make_async_copy(g_hbm.at[src_v.at[j]], msg0, sem0).wait()
        pltpu.sync_copy(msg0, acc_sh.at[dst_v.at[j]], add=True)
        pltpu.async_copy(g_hbm.at[src_v.at[j + 2]], msg0, sem0)
        pltpu.make_async_copy(g_hbm.at[src_v.at[j + 1]], msg1, sem1).wait()
        pltpu.sync_copy(msg1, acc_sh.at[dst_v.at[j + 1]], add=True)
        pltpu.async_copy(g_hbm.at[src_v.at[j + 3]], msg1, sem1)
        return carry

    lax.fori_loop(0, NCH // 2 - 1, body, 0)
    j = NCH - 2
    pltpu.make_async_copy(g_hbm.at[src_v.at[j]], msg0, sem0).wait()
    pltpu.sync_copy(msg0, acc_sh.at[dst_v.at[j]], add=True)
    pltpu.make_async_copy(g_hbm.at[src_v.at[j + 1]], msg1, sem1).wait()
    pltpu.sync_copy(msg1, acc_sh.at[dst_v.at[j + 1]], add=True)

    plsc.subcore_barrier()
    pltpu.sync_copy(acc_sh.at[pl.ds(sid * RPT, RPT)],
                    acc_out.at[cid, pl.ds(sid * RPT, RPT)])


def _tc_final_body(a0_ref, a1_ref, g_ref, dis_ref, b_ref, out_ref):
    s = a0_ref[...] + a1_ref[...] + g_ref[...]
    out_ref[...] = jnp.maximum(dis_ref[...] * s + b_ref[...], 0.0)


def kernel(x, edge_index, ptr, W, b):
    del ptr
    src = edge_index[0].astype(jnp.int32).reshape(NW, NCH, CH)
    dst = edge_index[1].astype(jnp.int32).reshape(NW, NCH, CH)
    zeros1d = jnp.zeros((NPAD,), jnp.float32)
    zeros2d = jnp.zeros((128, D_HID), jnp.float32)

    deg_parts = _sc_degree(dst, zeros1d)
    d0 = deg_parts[0, :N].reshape(N, 1)
    d1 = deg_parts[1, :N].reshape(N, 1)

    blk = 1000
    grid = N // blk
    g, dis = pl.pallas_call(
        _tc_prescale_body,
        grid=(grid,),
        in_specs=[
            pl.BlockSpec((blk, D_IN), lambda i: (i, 0)),
            pl.BlockSpec((D_IN, D_HID), lambda i: (0, 0)),
            pl.BlockSpec((blk, 1), lambda i: (i, 0)),
            pl.BlockSpec((blk, 1), lambda i: (i, 0)),
        ],
        out_specs=[
            pl.BlockSpec((blk, D_HID), lambda i: (i, 0)),
            pl.BlockSpec((blk, 1), lambda i: (i, 0)),
        ],
        out_shape=[
            jax.ShapeDtypeStruct((N, D_HID), jnp.float32),
            jax.ShapeDtypeStruct((N, 1), jnp.float32),
        ],
    )(x, W, d0, d1)

    acc_parts = _sc_messages(src, dst, g, zeros2d)
    a0 = acc_parts[0, :N]
    a1 = acc_parts[1, :N]

    out = pl.pallas_call(
        _tc_final_body,
        grid=(grid,),
        in_specs=[
            pl.BlockSpec((blk, D_HID), lambda i: (i, 0)),
            pl.BlockSpec((blk, D_HID), lambda i: (i, 0)),
            pl.BlockSpec((blk, D_HID), lambda i: (i, 0)),
            pl.BlockSpec((blk, 1), lambda i: (i, 0)),
            pl.BlockSpec((1, D_HID), lambda i: (0, 0)),
        ],
        out_specs=pl.BlockSpec((blk, D_HID), lambda i: (i, 0)),
        out_shape=jax.ShapeDtypeStruct((N, D_HID), jnp.float32),
    )(a0, a1, g, dis, b.reshape(1, D_HID))
    return out


# speedup vs baseline: 50.3713x; 1.0129x over previous
import functools

import jax
import jax.numpy as jnp
from jax import lax
from jax.experimental import pallas as pl
from jax.experimental.pallas import tpu as pltpu
from jax.experimental.pallas import tpu_sc as plsc

N = 10000
E = 320000
D_IN = 128
D_HID = 32

NC = 2
NS = 16
NW = NC * NS
CH = 120
NCH = 86
E2 = NW * NCH * CH
EPT = NCH * CH
NPAD = 10240
PAD_NODE = N
RPT = NPAD // NS
FW = 25
FR = N // FW

_mesh = plsc.VectorSubcoreMesh(core_axis_name="c", subcore_axis_name="s",
                               num_cores=NC, num_subcores=NS)
_sc_params = pltpu.CompilerParams(use_tc_tiling_on_sc=False)


@functools.partial(
    pl.kernel,
    out_type=jax.ShapeDtypeStruct((NC, N, 8), jnp.float32),
    mesh=_mesh,
    compiler_params=_sc_params,
    scratch_types=[
        pltpu.VMEM((NCH, CH), jnp.int32),
        pltpu.VMEM((128, 8), jnp.float32),
        pltpu.VMEM_SHARED((NPAD, 8), jnp.float32),
    ],
)
def _sc_degree(dst_hbm, ones_hbm, zeros_hbm, deg_out, dst_v, ones_v, deg_sh):
    cid = lax.axis_index("c")
    sid = lax.axis_index("s")
    wid = cid * NS + sid
    pltpu.sync_copy(dst_hbm.at[wid], dst_v)
    pltpu.sync_copy(ones_hbm, ones_v)
    pltpu.sync_copy(zeros_hbm, deg_sh.at[pl.ds(sid * RPT, RPT)])
    plsc.subcore_barrier()

    def body(j, carry):
        pltpu.sync_copy(ones_v.at[pl.ds(0, CH)], deg_sh.at[dst_v.at[j]],
                        add=True)
        return carry

    lax.fori_loop(0, NCH, body, 0)
    plsc.subcore_barrier()
    pltpu.sync_copy(deg_sh.at[pl.ds(sid * 625, 625)],
                    deg_out.at[cid, pl.ds(sid * 625, 625)])


def _tc_prescale_body(x_ref, w_ref, dp_ref, g_ref, dis_ref):
    deg = dp_ref[0, :, 0:1] + dp_ref[1, :, 0:1]
    dis = lax.rsqrt(deg)
    h = jnp.dot(x_ref[...], w_ref[...], preferred_element_type=jnp.float32)
    g_ref[...] = h * dis
    dis_ref[...] = jnp.broadcast_to(dis, dis_ref.shape)


@functools.partial(
    pl.kernel,
    out_type=jax.ShapeDtypeStruct((NC, N, D_HID), jnp.float32),
    mesh=_mesh,
    compiler_params=_sc_params,
    scratch_types=[
        pltpu.VMEM((NCH, CH), jnp.int32),
        pltpu.VMEM((NCH, CH), jnp.int32),
        pltpu.VMEM((CH, D_HID), jnp.float32),
        pltpu.VMEM((CH, D_HID), jnp.float32),
        pltpu.VMEM((128, D_HID), jnp.float32),
        pltpu.VMEM_SHARED((NPAD, D_HID), jnp.float32),
        pltpu.SemaphoreType.DMA,
        pltpu.SemaphoreType.DMA,
    ],
)
def _sc_messages(src_hbm, dst_hbm, g_hbm, zeros_hbm, acc_out,
                 src_v, dst_v, msg0, msg1, zb, acc_sh, sem0, sem1):
    cid = lax.axis_index("c")
    sid = lax.axis_index("s")
    wid = cid * NS + sid
    pltpu.sync_copy(src_hbm.at[wid], src_v)
    pltpu.sync_copy(dst_hbm.at[wid], dst_v)
    pltpu.sync_copy(zeros_hbm, zb)
    for jj in range(RPT // 128):
        pltpu.sync_copy(zb, acc_sh.at[pl.ds(sid * RPT + jj * 128, 128)])
    plsc.subcore_barrier()

    pltpu.async_copy(g_hbm.at[src_v.at[0]], msg0, sem0)
    pltpu.async_copy(g_hbm.at[src_v.at[1]], msg1, sem1)

    def body(j2, carry):
        j = j2 * 2
        pltpu.make_async_copy(g_hbm.at[src_v.at[j]], msg0, sem0).wait()
        pltpu.sync_copy(msg0, acc_sh.at[dst_v.at[j]], add=True)
        pltpu.async_copy(g_hbm.at[src_v.at[j + 2]], msg0, sem0)
        pltpu.make_async_copy(g_hbm.at[src_v.at[j + 1]], msg1, sem1).wait()
        pltpu.sync_copy(msg1, acc_sh.at[dst_v.at[j + 1]], add=True)
        pltpu.async_copy(g_hbm.at[src_v.at[j + 3]], msg1, sem1)
        return carry

    lax.fori_loop(0, NCH // 2 - 1, body, 0)
    j = NCH - 2
    pltpu.make_async_copy(g_hbm.at[src_v.at[j]], msg0, sem0).wait()
    pltpu.sync_copy(msg0, acc_sh.at[dst_v.at[j]], add=True)
    pltpu.make_async_copy(g_hbm.at[src_v.at[j + 1]], msg1, sem1).wait()
    pltpu.sync_copy(msg1, acc_sh.at[dst_v.at[j + 1]], add=True)

    plsc.subcore_barrier()
    pltpu.sync_copy(acc_sh.at[pl.ds(sid * 625, 625)],
                    acc_out.at[cid, pl.ds(sid * 625, 625)])


@functools.partial(
    pl.kernel,
    out_type=jax.ShapeDtypeStruct((N, D_HID), jnp.float32),
    mesh=_mesh,
    compiler_params=_sc_params,
    scratch_types=[
        pltpu.VMEM((FR, D_HID), jnp.float32),
        pltpu.VMEM((FR, D_HID), jnp.float32),
        pltpu.VMEM((FR, 16), jnp.float32),
        pltpu.VMEM((D_HID,), jnp.float32),
    ],
)
def _sc_final(acc_hbm, dis_hbm, b_hbm, out_hbm, a0_v, a1_v, dis_v, b_v):
    cid = lax.axis_index("c")
    sid = lax.axis_index("s")
    wid = cid * NS + sid

    @pl.when(wid < FW)
    def _():
        base = wid * FR
        pltpu.sync_copy(acc_hbm.at[0, pl.ds(base, FR)], a0_v)
        pltpu.sync_copy(acc_hbm.at[1, pl.ds(base, FR)], a1_v)
        pltpu.sync_copy(dis_hbm.at[pl.ds(base, FR)], dis_v)
        pltpu.sync_copy(b_hbm, b_v)
        bv0 = b_v[pl.ds(0, 16)]
        bv1 = b_v[pl.ds(16, 16)]

        def row(r, carry):
            dv = dis_v[r]
            v0 = (a0_v[r, pl.ds(0, 16)] + a1_v[r, pl.ds(0, 16)]) * dv + bv0
            v1 = (a0_v[r, pl.ds(16, 16)] + a1_v[r, pl.ds(16, 16)]) * dv + bv1
            a0_v[r, pl.ds(0, 16)] = jnp.maximum(v0, 0.0)
            a0_v[r, pl.ds(16, 16)] = jnp.maximum(v1, 0.0)
            return carry

        lax.fori_loop(0, FR, row, 0)
        pltpu.sync_copy(a0_v, out_hbm.at[pl.ds(base, FR)])


def kernel(x, edge_index, ptr, W, b):
    del ptr
    loop = jnp.arange(N, dtype=jnp.int32)
    pad = E2 - E - N
    src = jnp.concatenate([edge_index[0].astype(jnp.int32), loop,
                           jnp.zeros((pad,), jnp.int32)]).reshape(NW, NCH, CH)
    dst = jnp.concatenate([edge_index[1].astype(jnp.int32), loop,
                           jnp.full((pad,), PAD_NODE, jnp.int32)]
                          ).reshape(NW, NCH, CH)
    ones8 = jnp.ones((128, 8), jnp.float32)
    zeros8 = jnp.zeros((RPT, 8), jnp.float32)
    zeros2d = jnp.zeros((128, D_HID), jnp.float32)

    deg_parts = _sc_degree(dst, ones8, zeros8)

    blk = 1000
    grid = N // blk
    g, dis = pl.pallas_call(
        _tc_prescale_body,
        grid=(grid,),
        in_specs=[
            pl.BlockSpec((blk, D_IN), lambda i: (i, 0)),
            pl.BlockSpec((D_IN, D_HID), lambda i: (0, 0)),
            pl.BlockSpec((NC, blk, 8), lambda i: (0, i, 0)),
        ],
        out_specs=[
            pl.BlockSpec((blk, D_HID), lambda i: (i, 0)),
            pl.BlockSpec((blk, 16), lambda i: (i, 0)),
        ],
        out_shape=[
            jax.ShapeDtypeStruct((N, D_HID), jnp.float32),
            jax.ShapeDtypeStruct((N, 16), jnp.float32),
        ],
    )(x, W, deg_parts)

    acc_parts = _sc_messages(src, dst, g, zeros2d)
    return _sc_final(acc_parts, dis, b)


# speedup vs baseline: 54.3810x; 1.0796x over previous
import functools

import jax
import jax.numpy as jnp
from jax import lax
from jax.experimental import pallas as pl
from jax.experimental.pallas import tpu as pltpu
from jax.experimental.pallas import tpu_sc as plsc

N = 10000
E = 320000
D_IN = 128
D_HID = 32

NC = 2
NS = 16
EW = 25
CH = 128
NCH = 100
EPT = NCH * CH
NPAD = 10240
RPT = NPAD // NS
FW = 25
FR = N // FW

_mesh = plsc.VectorSubcoreMesh(core_axis_name="c", subcore_axis_name="s",
                               num_cores=NC, num_subcores=NS)
_sc_params = pltpu.CompilerParams(use_tc_tiling_on_sc=False)


def _tc_matmul_body(x_ref, w_ref, h_ref):
    h_ref[...] = jnp.dot(x_ref[...], w_ref[...],
                         preferred_element_type=jnp.float32)


@functools.partial(
    pl.kernel,
    out_type=jax.ShapeDtypeStruct((NC, N, 16), jnp.float32),
    mesh=_mesh,
    compiler_params=_sc_params,
    scratch_types=[
        pltpu.VMEM((NCH, CH), jnp.int32),
        pltpu.VMEM((CH, 16), jnp.float32),
        pltpu.VMEM_SHARED((NPAD, 16), jnp.float32),
    ],
)
def _sc_degree(ei_hbm, ones_hbm, zeros_hbm, deg_out, dst_v, ones_v, deg_sh):
    cid = lax.axis_index("c")
    sid = lax.axis_index("s")
    active = sid < 13 - cid
    aw = cid * 13 + sid

    @pl.when(active)
    def _():
        pltpu.sync_copy(ei_hbm.at[1, pl.ds(aw * NCH, NCH)], dst_v)
        pltpu.sync_copy(ones_hbm, ones_v)

    pltpu.sync_copy(zeros_hbm, deg_sh.at[pl.ds(sid * RPT, RPT)])
    plsc.subcore_barrier()

    @pl.when(active)
    def _():
        def body(j, carry):
            pltpu.sync_copy(ones_v, deg_sh.at[dst_v.at[j]], add=True)
            return carry

        lax.fori_loop(0, NCH, body, 0)

    plsc.subcore_barrier()
    pltpu.sync_copy(deg_sh.at[pl.ds(sid * 625, 625)],
                    deg_out.at[cid, pl.ds(sid * 625, 625)])


@functools.partial(
    pl.kernel,
    out_type=[jax.ShapeDtypeStruct((N, D_HID), jnp.float32),
              jax.ShapeDtypeStruct((N, 16), jnp.float32)],
    mesh=_mesh,
    compiler_params=_sc_params,
    scratch_types=[
        pltpu.VMEM((FR, D_HID), jnp.float32),
        pltpu.VMEM((FR, 16), jnp.float32),
        pltpu.VMEM((FR, 16), jnp.float32),
    ],
)
def _sc_prescale(h_hbm, degp_hbm, g_out, dis_out, h_v, p0_v, p1_v):
    cid = lax.axis_index("c")
    sid = lax.axis_index("s")
    wid = cid * NS + sid

    @pl.when(wid < FW)
    def _():
        base = wid * FR
        pltpu.sync_copy(h_hbm.at[pl.ds(base, FR)], h_v)
        pltpu.sync_copy(degp_hbm.at[0, pl.ds(base, FR)], p0_v)
        pltpu.sync_copy(degp_hbm.at[1, pl.ds(base, FR)], p1_v)

        def row(r, carry):
            deg = p0_v[r] + p1_v[r] + 1.0
            i = lax.bitcast_convert_type(deg, jnp.int32)
            i = jnp.full((16,), 0x5F3759DF, jnp.int32) - (i >> 1)
            y = lax.bitcast_convert_type(i, jnp.float32)
            y = y * (1.5 - 0.5 * deg * y * y)
            y = y * (1.5 - 0.5 * deg * y * y)
            y = y * (1.5 - 0.5 * deg * y * y)
            h_v[r, pl.ds(0, 16)] = h_v[r, pl.ds(0, 16)] * y
            h_v[r, pl.ds(16, 16)] = h_v[r, pl.ds(16, 16)] * y
            p1_v[r] = y
            return carry

        lax.fori_loop(0, FR, row, 0)
        pltpu.sync_copy(h_v, g_out.at[pl.ds(base, FR)])
        pltpu.sync_copy(p1_v, dis_out.at[pl.ds(base, FR)])


@functools.partial(
    pl.kernel,
    out_type=jax.ShapeDtypeStruct((NC, N, D_HID), jnp.float32),
    mesh=_mesh,
    compiler_params=_sc_params,
    scratch_types=[
        pltpu.VMEM((NCH, CH), jnp.int32),
        pltpu.VMEM((NCH, CH), jnp.int32),
        pltpu.VMEM((CH, D_HID), jnp.float32),
        pltpu.VMEM((CH, D_HID), jnp.float32),
        pltpu.VMEM((128, D_HID), jnp.float32),
        pltpu.VMEM_SHARED((NPAD, D_HID), jnp.float32),
        pltpu.SemaphoreType.DMA,
        pltpu.SemaphoreType.DMA,
    ],
)
def _sc_messages(ei_hbm, g_hbm, zeros_hbm, acc_out,
                 src_v, dst_v, msg0, msg1, zb, acc_sh, sem0, sem1):
    cid = lax.axis_index("c")
    sid = lax.axis_index("s")
    active = sid < 13 - cid
    aw = cid * 13 + sid

    @pl.when(active)
    def _():
        pltpu.sync_copy(ei_hbm.at[0, pl.ds(aw * NCH, NCH)], src_v)
        pltpu.sync_copy(ei_hbm.at[1, pl.ds(aw * NCH, NCH)], dst_v)

    pltpu.sync_copy(zeros_hbm, zb)
    for jj in range(RPT // 128):
        pltpu.sync_copy(zb, acc_sh.at[pl.ds(sid * RPT + jj * 128, 128)])
    plsc.subcore_barrier()

    @pl.when(active)
    def _():
        pltpu.async_copy(g_hbm.at[src_v.at[0]], msg0, sem0)
        pltpu.async_copy(g_hbm.at[src_v.at[1]], msg1, sem1)

        def body(j2, carry):
            j = j2 * 2
            pltpu.make_async_copy(g_hbm.at[src_v.at[j]], msg0, sem0).wait()
            pltpu.sync_copy(msg0, acc_sh.at[dst_v.at[j]], add=True)
            pltpu.async_copy(g_hbm.at[src_v.at[j + 2]], msg0, sem0)
            pltpu.make_async_copy(g_hbm.at[src_v.at[j + 1]], msg1,
                                  sem1).wait()
            pltpu.sync_copy(msg1, acc_sh.at[dst_v.at[j + 1]], add=True)
            pltpu.async_copy(g_hbm.at[src_v.at[j + 3]], msg1, sem1)
            return carry

        lax.fori_loop(0, NCH // 2 - 1, body, 0)
        j = NCH - 2
        pltpu.make_async_copy(g_hbm.at[src_v.at[j]], msg0, sem0).wait()
        pltpu.sync_copy(msg0, acc_sh.at[dst_v.at[j]], add=True)
        pltpu.make_async_copy(g_hbm.at[src_v.at[j + 1]], msg1, sem1).wait()
        pltpu.sync_copy(msg1, acc_sh.at[dst_v.at[j + 1]], add=True)

    plsc.subcore_barrier()
    pltpu.sync_copy(acc_sh.at[pl.ds(sid * 625, 625)],
                    acc_out.at[cid, pl.ds(sid * 625, 625)])


@functools.partial(
    pl.kernel,
    out_type=jax.ShapeDtypeStruct((N, D_HID), jnp.float32),
    mesh=_mesh,
    compiler_params=_sc_params,
    scratch_types=[
        pltpu.VMEM((FR, D_HID), jnp.float32),
        pltpu.VMEM((FR, D_HID), jnp.float32),
        pltpu.VMEM((FR, D_HID), jnp.float32),
        pltpu.VMEM((FR, 16), jnp.float32),
        pltpu.VMEM((D_HID,), jnp.float32),
    ],
)
def _sc_final(acc_hbm, dis_hbm, g_hbm, b_hbm, out_hbm,
              a0_v, a1_v, g_v, dis_v, b_v):
    cid = lax.axis_index("c")
    sid = lax.axis_index("s")
    wid = cid * NS + sid

    @pl.when(wid < FW)
    def _():
        base = wid * FR
        pltpu.sync_copy(acc_hbm.at[0, pl.ds(base, FR)], a0_v)
        pltpu.sync_copy(acc_hbm.at[1, pl.ds(base, FR)], a1_v)
        pltpu.sync_copy(g_hbm.at[pl.ds(base, FR)], g_v)
        pltpu.sync_copy(dis_hbm.at[pl.ds(base, FR)], dis_v)
        pltpu.sync_copy(b_hbm, b_v)
        bv0 = b_v[pl.ds(0, 16)]
        bv1 = b_v[pl.ds(16, 16)]

        def row(r, carry):
            dv = dis_v[r]
            v0 = (a0_v[r, pl.ds(0, 16)] + a1_v[r, pl.ds(0, 16)]
                  + g_v[r, pl.ds(0, 16)]) * dv + bv0
            v1 = (a0_v[r, pl.ds(16, 16)] + a1_v[r, pl.ds(16, 16)]
                  + g_v[r, pl.ds(16, 16)]) * dv + bv1
            a0_v[r, pl.ds(0, 16)] = jnp.maximum(v0, 0.0)
            a0_v[r, pl.ds(16, 16)] = jnp.maximum(v1, 0.0)
            return carry

        lax.fori_loop(0, FR, row, 0)
        pltpu.sync_copy(a0_v, out_hbm.at[pl.ds(base, FR)])


def kernel(x, edge_index, ptr, W, b):
    del ptr
    ei3 = edge_index.astype(jnp.int32).reshape(2, E // CH, CH)
    ones16 = jnp.ones((CH, 16), jnp.float32)
    zeros16 = jnp.zeros((RPT, 16), jnp.float32)
    zeros2d = jnp.zeros((128, D_HID), jnp.float32)

    blk = 2000
    h = pl.pallas_call(
        _tc_matmul_body,
        grid=(N // blk,),
        in_specs=[
            pl.BlockSpec((blk, D_IN), lambda i: (i, 0)),
            pl.BlockSpec((D_IN, D_HID), lambda i: (0, 0)),
        ],
        out_specs=pl.BlockSpec((blk, D_HID), lambda i: (i, 0)),
        out_shape=jax.ShapeDtypeStruct((N, D_HID), jnp.float32),
    )(x, W)

    deg_parts = _sc_degree(ei3, ones16, zeros16)
    g, dis16 = _sc_prescale(h, deg_parts)
    acc_parts = _sc_messages(ei3, g, zeros2d)
    return _sc_final(acc_parts, dis16, g, b)


# speedup vs baseline: 57.5030x; 1.0574x over previous
import functools

import jax
import jax.numpy as jnp
from jax import lax
from jax.experimental import pallas as pl
from jax.experimental.pallas import tpu as pltpu
from jax.experimental.pallas import tpu_sc as plsc

N = 10000
E = 320000
D_IN = 128
D_HID = 32

NC = 2
NS = 16
NW = NC * NS
CH = 125
NCH = 80
EPT = NCH * CH
NPAD = 10240
RPT = NPAD // NS
FW = 25
FR = N // FW

_mesh = plsc.VectorSubcoreMesh(core_axis_name="c", subcore_axis_name="s",
                               num_cores=NC, num_subcores=NS)
_sc_params = pltpu.CompilerParams(use_tc_tiling_on_sc=False)


def _tc_matmul_body(x_ref, w_ref, h_ref):
    h_ref[...] = jnp.dot(x_ref[...], w_ref[...],
                         preferred_element_type=jnp.float32)


@functools.partial(
    pl.kernel,
    out_type=jax.ShapeDtypeStruct((NC, N, 16), jnp.float32),
    mesh=_mesh,
    compiler_params=_sc_params,
    scratch_types=[
        pltpu.VMEM((NCH, CH), jnp.int32),
        pltpu.VMEM((128, 16), jnp.float32),
        pltpu.VMEM_SHARED((NPAD, 16), jnp.float32),
    ],
)
def _sc_degree(ei_hbm, ones_hbm, zeros_hbm, deg_out, dst_v, ones_v, deg_sh):
    cid = lax.axis_index("c")
    sid = lax.axis_index("s")
    wid = cid * NS + sid

    pltpu.sync_copy(ei_hbm.at[1, wid], dst_v)
    pltpu.sync_copy(ones_hbm, ones_v)
    pltpu.sync_copy(zeros_hbm, deg_sh.at[pl.ds(sid * RPT, RPT)])
    plsc.subcore_barrier()

    def body(j, carry):
        pltpu.sync_copy(ones_v.at[pl.ds(0, CH)], deg_sh.at[dst_v.at[j]],
                        add=True)
        return carry

    lax.fori_loop(0, NCH, body, 0)
    plsc.subcore_barrier()
    pltpu.sync_copy(deg_sh.at[pl.ds(sid * 625, 625)],
                    deg_out.at[cid, pl.ds(sid * 625, 625)])


@functools.partial(
    pl.kernel,
    out_type=[jax.ShapeDtypeStruct((N, D_HID), jnp.float32),
              jax.ShapeDtypeStruct((N, 16), jnp.float32)],
    mesh=_mesh,
    compiler_params=_sc_params,
    scratch_types=[
        pltpu.VMEM((FR, D_HID), jnp.float32),
        pltpu.VMEM((FR, 16), jnp.float32),
        pltpu.VMEM((FR, 16), jnp.float32),
    ],
)
def _sc_prescale(h_hbm, degp_hbm, g_out, dis_out, h_v, p0_v, p1_v):
    cid = lax.axis_index("c")
    sid = lax.axis_index("s")
    wid = cid * NS + sid

    @pl.when(wid < FW)
    def _():
        base = wid * FR
        pltpu.sync_copy(h_hbm.at[pl.ds(base, FR)], h_v)
        pltpu.sync_copy(degp_hbm.at[0, pl.ds(base, FR)], p0_v)
        pltpu.sync_copy(degp_hbm.at[1, pl.ds(base, FR)], p1_v)

        def row(r4, carry):
            for k in range(4):
                r = r4 * 4 + k
                deg = p0_v[r] + p1_v[r] + 1.0
                i = lax.bitcast_convert_type(deg, jnp.int32)
                i = jnp.full((16,), 0x5F3759DF, jnp.int32) - (i >> 1)
                y = lax.bitcast_convert_type(i, jnp.float32)
                y = y * (1.5 - 0.5 * deg * y * y)
                y = y * (1.5 - 0.5 * deg * y * y)
                y = y * (1.5 - 0.5 * deg * y * y)
                h_v[r, pl.ds(0, 16)] = h_v[r, pl.ds(0, 16)] * y
                h_v[r, pl.ds(16, 16)] = h_v[r, pl.ds(16, 16)] * y
                p1_v[r] = y
            return carry

        lax.fori_loop(0, FR // 4, row, 0)
        pltpu.sync_copy(h_v, g_out.at[pl.ds(base, FR)])
        pltpu.sync_copy(p1_v, dis_out.at[pl.ds(base, FR)])


@functools.partial(
    pl.kernel,
    out_type=jax.ShapeDtypeStruct((NC, N, D_HID), jnp.float32),
    mesh=_mesh,
    compiler_params=_sc_params,
    scratch_types=[
        pltpu.VMEM((NCH, CH), jnp.int32),
        pltpu.VMEM((NCH, CH), jnp.int32),
        pltpu.VMEM((CH, D_HID), jnp.float32),
        pltpu.VMEM((CH, D_HID), jnp.float32),
        pltpu.VMEM((128, D_HID), jnp.float32),
        pltpu.VMEM_SHARED((NPAD, D_HID), jnp.float32),
        pltpu.SemaphoreType.DMA,
        pltpu.SemaphoreType.DMA,
    ],
)
def _sc_messages(ei_hbm, g_hbm, zeros_hbm, acc_out,
                 src_v, dst_v, msg0, msg1, zb, acc_sh, sem0, sem1):
    cid = lax.axis_index("c")
    sid = lax.axis_index("s")
    wid = cid * NS + sid

    pltpu.sync_copy(ei_hbm.at[0, wid], src_v)
    pltpu.sync_copy(ei_hbm.at[1, wid], dst_v)
    pltpu.sync_copy(zeros_hbm, zb)
    for jj in range(RPT // 128):
        pltpu.sync_copy(zb, acc_sh.at[pl.ds(sid * RPT + jj * 128, 128)])
    plsc.subcore_barrier()

    pltpu.async_copy(g_hbm.at[src_v.at[0]], msg0, sem0)
    pltpu.async_copy(g_hbm.at[src_v.at[1]], msg1, sem1)

    def body(j2, carry):
        j = j2 * 2
        pltpu.make_async_copy(g_hbm.at[src_v.at[j]], msg0, sem0).wait()
        pltpu.sync_copy(msg0, acc_sh.at[dst_v.at[j]], add=True)
        pltpu.async_copy(g_hbm.at[src_v.at[j + 2]], msg0, sem0)
        pltpu.make_async_copy(g_hbm.at[src_v.at[j + 1]], msg1, sem1).wait()
        pltpu.sync_copy(msg1, acc_sh.at[dst_v.at[j + 1]], add=True)
        pltpu.async_copy(g_hbm.at[src_v.at[j + 3]], msg1, sem1)
        return carry

    lax.fori_loop(0, NCH // 2 - 1, body, 0)
    j = NCH - 2
    pltpu.make_async_copy(g_hbm.at[src_v.at[j]], msg0, sem0).wait()
    pltpu.sync_copy(msg0, acc_sh.at[dst_v.at[j]], add=True)
    pltpu.make_async_copy(g_hbm.at[src_v.at[j + 1]], msg1, sem1).wait()
    pltpu.sync_copy(msg1, acc_sh.at[dst_v.at[j + 1]], add=True)

    plsc.subcore_barrier()
    pltpu.sync_copy(acc_sh.at[pl.ds(sid * 625, 625)],
                    acc_out.at[cid, pl.ds(sid * 625, 625)])


@functools.partial(
    pl.kernel,
    out_type=jax.ShapeDtypeStruct((N, D_HID), jnp.float32),
    mesh=_mesh,
    compiler_params=_sc_params,
    scratch_types=[
        pltpu.VMEM((FR, D_HID), jnp.float32),
        pltpu.VMEM((FR, D_HID), jnp.float32),
        pltpu.VMEM((FR, D_HID), jnp.float32),
        pltpu.VMEM((FR, 16), jnp.float32),
        pltpu.VMEM((D_HID,), jnp.float32),
    ],
)
def _sc_final(acc_hbm, dis_hbm, g_hbm, b_hbm, out_hbm,
              a0_v, a1_v, g_v, dis_v, b_v):
    cid = lax.axis_index("c")
    sid = lax.axis_index("s")
    wid = cid * NS + sid

    @pl.when(wid < FW)
    def _():
        base = wid * FR
        pltpu.sync_copy(acc_hbm.at[0, pl.ds(base, FR)], a0_v)
        pltpu.sync_copy(acc_hbm.at[1, pl.ds(base, FR)], a1_v)
        pltpu.sync_copy(g_hbm.at[pl.ds(base, FR)], g_v)
        pltpu.sync_copy(dis_hbm.at[pl.ds(base, FR)], dis_v)
        pltpu.sync_copy(b_hbm, b_v)
        bv0 = b_v[pl.ds(0, 16)]
        bv1 = b_v[pl.ds(16, 16)]

        def row(r4, carry):
            for k in range(4):
                r = r4 * 4 + k
                dv = dis_v[r]
                v0 = (a0_v[r, pl.ds(0, 16)] + a1_v[r, pl.ds(0, 16)]
                      + g_v[r, pl.ds(0, 16)]) * dv + bv0
                v1 = (a0_v[r, pl.ds(16, 16)] + a1_v[r, pl.ds(16, 16)]
                      + g_v[r, pl.ds(16, 16)]) * dv + bv1
                a0_v[r, pl.ds(0, 16)] = jnp.maximum(v0, 0.0)
                a0_v[r, pl.ds(16, 16)] = jnp.maximum(v1, 0.0)
            return carry

        lax.fori_loop(0, FR // 4, row, 0)
        pltpu.sync_copy(a0_v, out_hbm.at[pl.ds(base, FR)])


def kernel(x, edge_index, ptr, W, b):
    del ptr
    ei4 = edge_index.astype(jnp.int32).reshape(2, NW, NCH, CH)
    ones16 = jnp.ones((128, 16), jnp.float32)
    zeros16 = jnp.zeros((RPT, 16), jnp.float32)
    zeros2d = jnp.zeros((128, D_HID), jnp.float32)

    blk = 2000
    h = pl.pallas_call(
        _tc_matmul_body,
        grid=(N // blk,),
        in_specs=[
            pl.BlockSpec((blk, D_IN), lambda i: (i, 0)),
            pl.BlockSpec((D_IN, D_HID), lambda i: (0, 0)),
        ],
        out_specs=pl.BlockSpec((blk, D_HID), lambda i: (i, 0)),
        out_shape=jax.ShapeDtypeStruct((N, D_HID), jnp.float32),
    )(x, W)

    deg_parts = _sc_degree(ei4, ones16, zeros16)
    g, dis16 = _sc_prescale(h, deg_parts)
    acc_parts = _sc_messages(ei4, g, zeros2d)
    return _sc_final(acc_parts, dis16, g, b)


# speedup vs baseline: 72.7524x; 1.2652x over previous
import functools

import jax
import jax.numpy as jnp
from jax import lax
from jax.experimental import pallas as pl
from jax.experimental.pallas import tpu as pltpu
from jax.experimental.pallas import tpu_sc as plsc

N = 10000
E = 320000
D_IN = 128
D_HID = 32

NC = 2
NS = 16
NW = NC * NS
CH = 125
NCH = 80
EPT = NCH * CH
NPAD = 10240
RPT = NPAD // NS
FW = 25
FR = N // FW

_mesh = plsc.VectorSubcoreMesh(core_axis_name="c", subcore_axis_name="s",
                               num_cores=NC, num_subcores=NS)
_sc_params = pltpu.CompilerParams(use_tc_tiling_on_sc=False)


def _tc_matmul_body(x_ref, w_ref, h_ref):
    h_ref[...] = jnp.dot(x_ref[...], w_ref[...],
                         preferred_element_type=jnp.float32)


@functools.partial(
    pl.kernel,
    out_type=jax.ShapeDtypeStruct((NC, N), jnp.float32),
    mesh=_mesh,
    compiler_params=_sc_params,
    scratch_types=[
        pltpu.VMEM((NCH, CH), jnp.int32),
        pltpu.VMEM((128,), jnp.float32),
        pltpu.VMEM_SHARED((NPAD,), jnp.float32),
        pltpu.SemaphoreType.DMA,
    ],
)
def _sc_degree(ei_hbm, ones_hbm, zeros_hbm, deg_out, dst_v, ones_v, deg_sh,
               sem):
    cid = lax.axis_index("c")
    sid = lax.axis_index("s")
    wid = cid * NS + sid

    pltpu.sync_copy(ei_hbm.at[1, wid], dst_v)
    pltpu.sync_copy(ones_hbm, ones_v)
    pltpu.sync_copy(zeros_hbm, deg_sh.at[pl.ds(sid * RPT, RPT)])
    plsc.subcore_barrier()

    W = 8
    src = ones_v.at[pl.ds(0, CH)]
    for j in range(W):
        pltpu.async_copy(src, deg_sh.at[dst_v.at[j]], sem, add=True)

    def body(j, carry):
        pltpu.make_async_copy(src, deg_sh.at[dst_v.at[j]], sem).wait()
        pltpu.async_copy(src, deg_sh.at[dst_v.at[j + W]], sem, add=True)
        return carry

    lax.fori_loop(0, NCH - W, body, 0)
    for j in range(NCH - W, NCH):
        pltpu.make_async_copy(src, deg_sh.at[dst_v.at[j]], sem).wait()

    plsc.subcore_barrier()

    @pl.when(sid < 10)
    def _():
        pltpu.sync_copy(deg_sh.at[pl.ds(sid * 1000, 1000)],
                        deg_out.at[cid, pl.ds(sid * 1000, 1000)])


@functools.partial(
    pl.kernel,
    out_type=[jax.ShapeDtypeStruct((N, D_HID), jnp.float32),
              jax.ShapeDtypeStruct((N, 16), jnp.float32)],
    mesh=_mesh,
    compiler_params=_sc_params,
    scratch_types=[
        pltpu.VMEM((FR, D_HID), jnp.float32),
        pltpu.VMEM((FR,), jnp.float32),
        pltpu.VMEM((FR,), jnp.float32),
        pltpu.VMEM((FR, 16), jnp.float32),
    ],
)
def _sc_prescale(h_hbm, degp_hbm, g_out, dis_out, h_v, p0_v, p1_v, dis_v):
    cid = lax.axis_index("c")
    sid = lax.axis_index("s")
    wid = cid * NS + sid

    @pl.when(wid < FW)
    def _():
        base = wid * FR
        pltpu.sync_copy(h_hbm.at[pl.ds(base, FR)], h_v)
        pltpu.sync_copy(degp_hbm.at[0, pl.ds(base, FR)], p0_v)
        pltpu.sync_copy(degp_hbm.at[1, pl.ds(base, FR)], p1_v)

        def grp(gi, carry):
            deg = p0_v[pl.ds(gi * 16, 16)] + p1_v[pl.ds(gi * 16, 16)] + 1.0
            i = lax.bitcast_convert_type(deg, jnp.int32)
            i = jnp.full((16,), 0x5F3759DF, jnp.int32) - (i >> 1)
            y = lax.bitcast_convert_type(i, jnp.float32)
            y = y * (1.5 - 0.5 * deg * y * y)
            y = y * (1.5 - 0.5 * deg * y * y)
            y = y * (1.5 - 0.5 * deg * y * y)
            for k in range(16):
                s = y[k]
                r = gi * 16 + k
                h_v[r, pl.ds(0, 16)] = h_v[r, pl.ds(0, 16)] * s
                h_v[r, pl.ds(16, 16)] = h_v[r, pl.ds(16, 16)] * s
                dis_v[r] = jnp.full((16,), s, jnp.float32)
            return carry

        lax.fori_loop(0, FR // 16, grp, 0)
        pltpu.sync_copy(h_v, g_out.at[pl.ds(base, FR)])
        pltpu.sync_copy(dis_v, dis_out.at[pl.ds(base, FR)])


@functools.partial(
    pl.kernel,
    out_type=jax.ShapeDtypeStruct((NC, N, D_HID), jnp.float32),
    mesh=_mesh,
    compiler_params=_sc_params,
    scratch_types=[
        pltpu.VMEM((NCH, CH), jnp.int32),
        pltpu.VMEM((NCH, CH), jnp.int32),
        pltpu.VMEM((CH, D_HID), jnp.float32),
        pltpu.VMEM((CH, D_HID), jnp.float32),
        pltpu.VMEM((CH, D_HID), jnp.float32),
        pltpu.VMEM((CH, D_HID), jnp.float32),
        pltpu.VMEM((128, D_HID), jnp.float32),
        pltpu.VMEM_SHARED((NPAD, D_HID), jnp.float32),
        pltpu.SemaphoreType.DMA,
        pltpu.SemaphoreType.DMA,
        pltpu.SemaphoreType.DMA,
        pltpu.SemaphoreType.DMA,
        pltpu.SemaphoreType.DMA,
        pltpu.SemaphoreType.DMA,
        pltpu.SemaphoreType.DMA,
        pltpu.SemaphoreType.DMA,
    ],
)
def _sc_messages(ei_hbm, g_hbm, zeros_hbm, acc_out,
                 src_v, dst_v, msg0, msg1, msg2, msg3, zb, acc_sh,
                 sg0, sg1, sg2, sg3, ss0, ss1, ss2, ss3):
    cid = lax.axis_index("c")
    sid = lax.axis_index("s")
    wid = cid * NS + sid
    msg = [msg0, msg1, msg2, msg3]
    sg = [sg0, sg1, sg2, sg3]
    ss = [ss0, ss1, ss2, ss3]

    pltpu.sync_copy(ei_hbm.at[0, wid], src_v)
    pltpu.sync_copy(ei_hbm.at[1, wid], dst_v)
    pltpu.sync_copy(zeros_hbm, zb)
    for jj in range(RPT // 128):
        pltpu.sync_copy(zb, acc_sh.at[pl.ds(sid * RPT + jj * 128, 128)])
    plsc.subcore_barrier()

    for b in range(4):
        pltpu.async_copy(g_hbm.at[src_v.at[b]], msg[b], sg[b])

    def quad(q, carry):
        j0 = q * 4
        for b in range(4):
            j = j0 + b
            pltpu.make_async_copy(g_hbm.at[src_v.at[j]], msg[b],
                                  sg[b]).wait()
            pltpu.async_copy(msg[b], acc_sh.at[dst_v.at[j]], ss[b],
                             add=True)
        for b in range(4):
            j = j0 + b
            pltpu.make_async_copy(msg[b], acc_sh.at[dst_v.at[j]],
                                  ss[b]).wait()
            pltpu.async_copy(g_hbm.at[src_v.at[j + 4]], msg[b], sg[b])
        return carry

    lax.fori_loop(0, NCH // 4 - 1, quad, 0)
    j0 = NCH - 4
    for b in range(4):
        j = j0 + b
        pltpu.make_async_copy(g_hbm.at[src_v.at[j]], msg[b], sg[b]).wait()
        pltpu.async_copy(msg[b], acc_sh.at[dst_v.at[j]], ss[b], add=True)
    for b in range(4):
        j = j0 + b
        pltpu.make_async_copy(msg[b], acc_sh.at[dst_v.at[j]], ss[b]).wait()

    plsc.subcore_barrier()
    pltpu.sync_copy(acc_sh.at[pl.ds(sid * 625, 625)],
                    acc_out.at[cid, pl.ds(sid * 625, 625)])


@functools.partial(
    pl.kernel,
    out_type=jax.ShapeDtypeStruct((N, D_HID), jnp.float32),
    mesh=_mesh,
    compiler_params=_sc_params,
    scratch_types=[
        pltpu.VMEM((FR, D_HID), jnp.float32),
        pltpu.VMEM((FR, D_HID), jnp.float32),
        pltpu.VMEM((FR, D_HID), jnp.float32),
        pltpu.VMEM((FR, 16), jnp.float32),
        pltpu.VMEM((D_HID,), jnp.float32),
    ],
)
def _sc_final(acc_hbm, dis_hbm, g_hbm, b_hbm, out_hbm,
              a0_v, a1_v, g_v, dis_v, b_v):
    cid = lax.axis_index("c")
    sid = lax.axis_index("s")
    wid = cid * NS + sid

    @pl.when(wid < FW)
    def _():
        base = wid * FR
        pltpu.sync_copy(acc_hbm.at[0, pl.ds(base, FR)], a0_v)
        pltpu.sync_copy(acc_hbm.at[1, pl.ds(base, FR)], a1_v)
        pltpu.sync_copy(g_hbm.at[pl.ds(base, FR)], g_v)
        pltpu.sync_copy(dis_hbm.at[pl.ds(base, FR)], dis_v)
        pltpu.sync_copy(b_hbm, b_v)
        bv0 = b_v[pl.ds(0, 16)]
        bv1 = b_v[pl.ds(16, 16)]

        def row(r4, carry):
            for k in range(4):
                r = r4 * 4 + k
                dv = dis_v[r]
                v0 = (a0_v[r, pl.ds(0, 16)] + a1_v[r, pl.ds(0, 16)]
                      + g_v[r, pl.ds(0, 16)]) * dv + bv0
                v1 = (a0_v[r, pl.ds(16, 16)] + a1_v[r, pl.ds(16, 16)]
                      + g_v[r, pl.ds(16, 16)]) * dv + bv1
                a0_v[r, pl.ds(0, 16)] = jnp.maximum(v0, 0.0)
                a0_v[r, pl.ds(16, 16)] = jnp.maximum(v1, 0.0)
            return carry

        lax.fori_loop(0, FR // 4, row, 0)
        pltpu.sync_copy(a0_v, out_hbm.at[pl.ds(base, FR)])


def kernel(x, edge_index, ptr, W, b):
    del ptr
    ei4 = edge_index.astype(jnp.int32).reshape(2, NW, NCH, CH)
    ones1 = jnp.ones((128,), jnp.float32)
    zeros1 = jnp.zeros((RPT,), jnp.float32)
    zeros2d = jnp.zeros((128, D_HID), jnp.float32)

    blk = 2000
    h = pl.pallas_call(
        _tc_matmul_body,
        grid=(N // blk,),
        in_specs=[
            pl.BlockSpec((blk, D_IN), lambda i: (i, 0)),
            pl.BlockSpec((D_IN, D_HID), lambda i: (0, 0)),
        ],
        out_specs=pl.BlockSpec((blk, D_HID), lambda i: (i, 0)),
        out_shape=jax.ShapeDtypeStruct((N, D_HID), jnp.float32),
    )(x, W)

    deg_parts = _sc_degree(ei4, ones1, zeros1)
    g, dis16 = _sc_prescale(h, deg_parts)
    acc_parts = _sc_messages(ei4, g, zeros2d)
    return _sc_final(acc_parts, dis16, g, b)


# speedup vs baseline: 76.6596x; 1.0537x over previous
import functools

import jax
import jax.numpy as jnp
from jax import lax
from jax.experimental import pallas as pl
from jax.experimental.pallas import tpu as pltpu
from jax.experimental.pallas import tpu_sc as plsc

N = 10000
E = 320000
D_IN = 128
D_HID = 32

NC = 2
NS = 16
NW = NC * NS
EPT = E // NW
CH = 128
NCH = EPT // CH
TL = EPT - NCH * CH
NPAD = 10240
RPT = NPAD // NS
FW = 25
FR = N // FW

_mesh = plsc.VectorSubcoreMesh(core_axis_name="c", subcore_axis_name="s",
                               num_cores=NC, num_subcores=NS)
_sc_params = pltpu.CompilerParams(use_tc_tiling_on_sc=False)


def _tc_matmul_body(x_ref, w_ref, h_ref):
    h_ref[...] = jnp.dot(x_ref[...], w_ref[...],
                         preferred_element_type=jnp.float32)


@functools.partial(
    pl.kernel,
    out_type=jax.ShapeDtypeStruct((NC, N), jnp.float32),
    mesh=_mesh,
    compiler_params=_sc_params,
    scratch_types=[
        pltpu.VMEM((EPT,), jnp.int32),
        pltpu.VMEM((128,), jnp.float32),
        pltpu.VMEM_SHARED((NPAD,), jnp.float32),
        pltpu.SemaphoreType.DMA,
    ],
)
def _sc_degree(ei_hbm, ones_hbm, zeros_hbm, deg_out, dst_v, ones_v, deg_sh,
               sem):
    cid = lax.axis_index("c")
    sid = lax.axis_index("s")
    wid = cid * NS + sid

    pltpu.sync_copy(ei_hbm.at[1, pl.ds(wid * EPT, EPT)], dst_v)
    pltpu.sync_copy(ones_hbm, ones_v)
    pltpu.sync_copy(zeros_hbm, deg_sh.at[pl.ds(sid * RPT, RPT)])
    plsc.subcore_barrier()

    W = 8
    src = ones_v.at[pl.ds(0, CH)]

    def idx(j):
        return dst_v.at[pl.ds(j * CH, CH)]

    for j in range(W):
        pltpu.async_copy(src, deg_sh.at[idx(j)], sem, add=True)

    def body(j, carry):
        pltpu.make_async_copy(src, deg_sh.at[idx(j)], sem).wait()
        pltpu.async_copy(src, deg_sh.at[idx(j + W)], sem, add=True)
        return carry

    lax.fori_loop(0, NCH - W, body, 0)
    for j in range(NCH - W, NCH):
        pltpu.make_async_copy(src, deg_sh.at[idx(j)], sem).wait()
    tsrc = ones_v.at[pl.ds(0, TL)]
    tidx = dst_v.at[pl.ds(NCH * CH, TL)]
    pltpu.async_copy(tsrc, deg_sh.at[tidx], sem, add=True)
    pltpu.make_async_copy(tsrc, deg_sh.at[tidx], sem).wait()

    plsc.subcore_barrier()

    @pl.when(sid < 10)
    def _():
        pltpu.sync_copy(deg_sh.at[pl.ds(sid * 1000, 1000)],
                        deg_out.at[cid, pl.ds(sid * 1000, 1000)])


@functools.partial(
    pl.kernel,
    out_type=[jax.ShapeDtypeStruct((N, D_HID), jnp.float32),
              jax.ShapeDtypeStruct((N, 16), jnp.float32)],
    mesh=_mesh,
    compiler_params=_sc_params,
    scratch_types=[
        pltpu.VMEM((320, D_HID), jnp.float32),
        pltpu.VMEM((320,), jnp.float32),
        pltpu.VMEM((320,), jnp.float32),
        pltpu.VMEM((320, 16), jnp.float32),
    ],
)
def _sc_prescale(h_hbm, degp_hbm, g_out, dis_out, h_v, p0_v, p1_v, dis_v):
    cid = lax.axis_index("c")
    sid = lax.axis_index("s")
    wid = cid * NS + sid

    def run(nrows):
        base = wid * 320
        pltpu.sync_copy(h_hbm.at[pl.ds(base, nrows)],
                        h_v.at[pl.ds(0, nrows)])
        pltpu.sync_copy(degp_hbm.at[0, pl.ds(base, nrows)],
                        p0_v.at[pl.ds(0, nrows)])
        pltpu.sync_copy(degp_hbm.at[1, pl.ds(base, nrows)],
                        p1_v.at[pl.ds(0, nrows)])

        def grp(gi, carry):
            deg = p0_v[pl.ds(gi * 16, 16)] + p1_v[pl.ds(gi * 16, 16)] + 1.0
            i = lax.bitcast_convert_type(deg, jnp.int32)
            i = jnp.full((16,), 0x5F3759DF, jnp.int32) - (i >> 1)
            y = lax.bitcast_convert_type(i, jnp.float32)
            y = y * (1.5 - 0.5 * deg * y * y)
            y = y * (1.5 - 0.5 * deg * y * y)
            y = y * (1.5 - 0.5 * deg * y * y)
            for k in range(16):
                s = y[k]
                r = gi * 16 + k
                h_v[r, pl.ds(0, 16)] = h_v[r, pl.ds(0, 16)] * s
                h_v[r, pl.ds(16, 16)] = h_v[r, pl.ds(16, 16)] * s
                dis_v[r] = jnp.full((16,), s, jnp.float32)
            return carry

        lax.fori_loop(0, nrows // 16, grp, 0)
        pltpu.sync_copy(h_v.at[pl.ds(0, nrows)], g_out.at[pl.ds(base, nrows)])
        pltpu.sync_copy(dis_v.at[pl.ds(0, nrows)],
                        dis_out.at[pl.ds(base, nrows)])

    @pl.when(wid < NW - 1)
    def _():
        run(320)

    @pl.when(wid == NW - 1)
    def _():
        run(N - 320 * (NW - 1))


@functools.partial(
    pl.kernel,
    out_type=jax.ShapeDtypeStruct((NC, N, D_HID), jnp.float32),
    mesh=_mesh,
    compiler_params=_sc_params,
    scratch_types=[
        pltpu.VMEM((EPT,), jnp.int32),
        pltpu.VMEM((EPT,), jnp.int32),
        pltpu.VMEM((CH, D_HID), jnp.float32),
        pltpu.VMEM((CH, D_HID), jnp.float32),
        pltpu.VMEM((CH, D_HID), jnp.float32),
        pltpu.VMEM((CH, D_HID), jnp.float32),
        pltpu.VMEM((128, D_HID), jnp.float32),
        pltpu.VMEM_SHARED((NPAD, D_HID), jnp.float32),
        pltpu.SemaphoreType.DMA,
        pltpu.SemaphoreType.DMA,
        pltpu.SemaphoreType.DMA,
        pltpu.SemaphoreType.DMA,
        pltpu.SemaphoreType.DMA,
        pltpu.SemaphoreType.DMA,
        pltpu.SemaphoreType.DMA,
        pltpu.SemaphoreType.DMA,
    ],
)
def _sc_messages(ei_hbm, g_hbm, zeros_hbm, acc_out,
                 src_v, dst_v, msg0, msg1, msg2, msg3, zb, acc_sh,
                 sg0, sg1, sg2, sg3, ss0, ss1, ss2, ss3):
    cid = lax.axis_index("c")
    sid = lax.axis_index("s")
    wid = cid * NS + sid
    msg = [msg0, msg1, msg2, msg3]
    sg = [sg0, sg1, sg2, sg3]
    ss = [ss0, ss1, ss2, ss3]

    pltpu.sync_copy(ei_hbm.at[0, pl.ds(wid * EPT, EPT)], src_v)
    pltpu.sync_copy(ei_hbm.at[1, pl.ds(wid * EPT, EPT)], dst_v)
    pltpu.sync_copy(zeros_hbm, zb)
    for jj in range(RPT // 128):
        pltpu.sync_copy(zb, acc_sh.at[pl.ds(sid * RPT + jj * 128, 128)])
    plsc.subcore_barrier()

    def sidx(j):
        return src_v.at[pl.ds(j * CH, CH)]

    def didx(j):
        return dst_v.at[pl.ds(j * CH, CH)]

    for b in range(4):
        pltpu.async_copy(g_hbm.at[sidx(b)], msg[b], sg[b])

    def quad(q, carry):
        j0 = q * 4
        for b in range(4):
            j = j0 + b
            pltpu.make_async_copy(g_hbm.at[sidx(j)], msg[b], sg[b]).wait()
            pltpu.async_copy(msg[b], acc_sh.at[didx(j)], ss[b], add=True)
        for b in range(4):
            j = j0 + b
            pltpu.make_async_copy(msg[b], acc_sh.at[didx(j)], ss[b]).wait()
            pltpu.async_copy(g_hbm.at[sidx(j + 4)], msg[b], sg[b])
        return carry

    lax.fori_loop(0, NCH // 4 - 1, quad, 0)
    j0 = NCH - 6
    for b in range(4):
        j = j0 + b
        pltpu.make_async_copy(g_hbm.at[sidx(j)], msg[b], sg[b]).wait()
        pltpu.async_copy(msg[b], acc_sh.at[didx(j)], ss[b], add=True)
    for b in range(2):
        j = j0 + b
        pltpu.make_async_copy(msg[b], acc_sh.at[didx(j)], ss[b]).wait()
        pltpu.async_copy(g_hbm.at[sidx(j + 4)], msg[b], sg[b])
    for b in range(2):
        j = j0 + 4 + b
        pltpu.make_async_copy(g_hbm.at[sidx(j)], msg[b], sg[b]).wait()
        pltpu.async_copy(msg[b], acc_sh.at[didx(j)], ss[b], add=True)
        pltpu.make_async_copy(msg[b], acc_sh.at[didx(j)], ss[b]).wait()
    for b in range(2, 4):
        j = j0 + b
        pltpu.make_async_copy(msg[b], acc_sh.at[didx(j)], ss[b]).wait()
    tsi = src_v.at[pl.ds(NCH * CH, TL)]
    tdi = dst_v.at[pl.ds(NCH * CH, TL)]
    tmsg = msg2.at[pl.ds(0, TL)]
    pltpu.async_copy(g_hbm.at[tsi], tmsg, sg2)
    pltpu.make_async_copy(g_hbm.at[tsi], tmsg, sg2).wait()
    pltpu.async_copy(tmsg, acc_sh.at[tdi], ss2, add=True)
    pltpu.make_async_copy(tmsg, acc_sh.at[tdi], ss2).wait()

    plsc.subcore_barrier()
    pltpu.sync_copy(acc_sh.at[pl.ds(sid * 625, 625)],
                    acc_out.at[cid, pl.ds(sid * 625, 625)])


@functools.partial(
    pl.kernel,
    out_type=jax.ShapeDtypeStruct((N, D_HID), jnp.float32),
    mesh=_mesh,
    compiler_params=_sc_params,
    scratch_types=[
        pltpu.VMEM((320, D_HID), jnp.float32),
        pltpu.VMEM((320, D_HID), jnp.float32),
        pltpu.VMEM((320, D_HID), jnp.float32),
        pltpu.VMEM((320, 16), jnp.float32),
        pltpu.VMEM((D_HID,), jnp.float32),
    ],
)
def _sc_final(acc_hbm, dis_hbm, g_hbm, b_hbm, out_hbm,
              a0_v, a1_v, g_v, dis_v, b_v):
    cid = lax.axis_index("c")
    sid = lax.axis_index("s")
    wid = cid * NS + sid

    def run(nrows):
        base = wid * 320
        pltpu.sync_copy(acc_hbm.at[0, pl.ds(base, nrows)],
                        a0_v.at[pl.ds(0, nrows)])
        pltpu.sync_copy(acc_hbm.at[1, pl.ds(base, nrows)],
                        a1_v.at[pl.ds(0, nrows)])
        pltpu.sync_copy(g_hbm.at[pl.ds(base, nrows)],
                        g_v.at[pl.ds(0, nrows)])
        pltpu.sync_copy(dis_hbm.at[pl.ds(base, nrows)],
                        dis_v.at[pl.ds(0, nrows)])
        pltpu.sync_copy(b_hbm, b_v)
        bv0 = b_v[pl.ds(0, 16)]
        bv1 = b_v[pl.ds(16, 16)]

        def row(r4, carry):
            for k in range(4):
                r = r4 * 4 + k
                dv = dis_v[r]
                v0 = (a0_v[r, pl.ds(0, 16)] + a1_v[r, pl.ds(0, 16)]
                      + g_v[r, pl.ds(0, 16)]) * dv + bv0
                v1 = (a0_v[r, pl.ds(16, 16)] + a1_v[r, pl.ds(16, 16)]
                      + g_v[r, pl.ds(16, 16)]) * dv + bv1
                a0_v[r, pl.ds(0, 16)] = jnp.maximum(v0, 0.0)
                a0_v[r, pl.ds(16, 16)] = jnp.maximum(v1, 0.0)
            return carry

        lax.fori_loop(0, nrows // 4, row, 0)
        pltpu.sync_copy(a0_v.at[pl.ds(0, nrows)],
                        out_hbm.at[pl.ds(base, nrows)])

    @pl.when(wid < NW - 1)
    def _():
        run(320)

    @pl.when(wid == NW - 1)
    def _():
        run(N - 320 * (NW - 1))


def kernel(x, edge_index, ptr, W, b):
    del ptr
    ei = edge_index.astype(jnp.int32)
    ones1 = jnp.ones((128,), jnp.float32)
    zeros1 = jnp.zeros((RPT,), jnp.float32)
    zeros2d = jnp.zeros((128, D_HID), jnp.float32)

    blk = 2000
    h = pl.pallas_call(
        _tc_matmul_body,
        grid=(N // blk,),
        in_specs=[
            pl.BlockSpec((blk, D_IN), lambda i: (i, 0)),
            pl.BlockSpec((D_IN, D_HID), lambda i: (0, 0)),
        ],
        out_specs=pl.BlockSpec((blk, D_HID), lambda i: (i, 0)),
        out_shape=jax.ShapeDtypeStruct((N, D_HID), jnp.float32),
    )(x, W)

    deg_parts = _sc_degree(ei, ones1, zeros1)
    g, dis16 = _sc_prescale(h, deg_parts)
    acc_parts = _sc_messages(ei, g, zeros2d)
    return _sc_final(acc_parts, dis16, g, b)


# speedup vs baseline: 77.9233x; 1.0165x over previous
import functools

import jax
import jax.numpy as jnp
from jax import lax
from jax.experimental import pallas as pl
from jax.experimental.pallas import tpu as pltpu
from jax.experimental.pallas import tpu_sc as plsc

N = 10000
E = 320000
D_IN = 128
D_HID = 32

NC = 2
NS = 16
NW = NC * NS
EPT = E // NW
CH = 128
NCH = EPT // CH
TL = EPT - NCH * CH
NPAD = 10240
RPT = NPAD // NS
FW = 25
FR = N // FW

_mesh = plsc.VectorSubcoreMesh(core_axis_name="c", subcore_axis_name="s",
                               num_cores=NC, num_subcores=NS)
_sc_params = pltpu.CompilerParams(use_tc_tiling_on_sc=False)


def _tc_matmul_body(x_ref, w_ref, h_ref):
    h_ref[...] = jnp.dot(x_ref[...], w_ref[...],
                         preferred_element_type=jnp.float32)


@functools.partial(
    pl.kernel,
    out_type=jax.ShapeDtypeStruct((NC, N), jnp.float32),
    mesh=_mesh,
    compiler_params=_sc_params,
    scratch_types=[
        pltpu.VMEM((EPT,), jnp.int32),
        pltpu.VMEM((128,), jnp.float32),
        pltpu.VMEM((RPT,), jnp.float32),
        pltpu.VMEM_SHARED((NPAD,), jnp.float32),
        pltpu.SemaphoreType.DMA,
    ],
)
def _sc_degree(ei_hbm, deg_out, dst_v, ones_v, zeros_v, deg_sh, sem):
    cid = lax.axis_index("c")
    sid = lax.axis_index("s")
    wid = cid * NS + sid

    pltpu.sync_copy(ei_hbm.at[1, pl.ds(wid * EPT, EPT)], dst_v)
    for i in range(8):
        ones_v[pl.ds(i * 16, 16)] = jnp.full((16,), 1.0, jnp.float32)
    for i in range(RPT // 16):
        zeros_v[pl.ds(i * 16, 16)] = jnp.zeros((16,), jnp.float32)
    pltpu.sync_copy(zeros_v, deg_sh.at[pl.ds(sid * RPT, RPT)])
    plsc.subcore_barrier()

    W = 8
    src = ones_v.at[pl.ds(0, CH)]

    def idx(j):
        return dst_v.at[pl.ds(j * CH, CH)]

    for j in range(W):
        pltpu.async_copy(src, deg_sh.at[idx(j)], sem, add=True)

    def body(j, carry):
        pltpu.make_async_copy(src, deg_sh.at[idx(j)], sem).wait()
        pltpu.async_copy(src, deg_sh.at[idx(j + W)], sem, add=True)
        return carry

    lax.fori_loop(0, NCH - W, body, 0)
    for j in range(NCH - W, NCH):
        pltpu.make_async_copy(src, deg_sh.at[idx(j)], sem).wait()
    tsrc = ones_v.at[pl.ds(0, TL)]
    tidx = dst_v.at[pl.ds(NCH * CH, TL)]
    pltpu.async_copy(tsrc, deg_sh.at[tidx], sem, add=True)
    pltpu.make_async_copy(tsrc, deg_sh.at[tidx], sem).wait()

    plsc.subcore_barrier()

    @pl.when(sid < 10)
    def _():
        pltpu.sync_copy(deg_sh.at[pl.ds(sid * 1000, 1000)],
                        deg_out.at[cid, pl.ds(sid * 1000, 1000)])


@functools.partial(
    pl.kernel,
    out_type=[jax.ShapeDtypeStruct((N, D_HID), jnp.float32),
              jax.ShapeDtypeStruct((N, 16), jnp.float32)],
    mesh=_mesh,
    compiler_params=_sc_params,
    scratch_types=[
        pltpu.VMEM((320, D_HID), jnp.float32),
        pltpu.VMEM((320,), jnp.float32),
        pltpu.VMEM((320,), jnp.float32),
        pltpu.VMEM((320, 16), jnp.float32),
    ],
)
def _sc_prescale(h_hbm, degp_hbm, g_out, dis_out, h_v, p0_v, p1_v, dis_v):
    cid = lax.axis_index("c")
    sid = lax.axis_index("s")
    wid = cid * NS + sid

    def run(nrows):
        base = wid * 320
        pltpu.sync_copy(h_hbm.at[pl.ds(base, nrows)],
                        h_v.at[pl.ds(0, nrows)])
        pltpu.sync_copy(degp_hbm.at[0, pl.ds(base, nrows)],
                        p0_v.at[pl.ds(0, nrows)])
        pltpu.sync_copy(degp_hbm.at[1, pl.ds(base, nrows)],
                        p1_v.at[pl.ds(0, nrows)])

        def grp(gi, carry):
            deg = p0_v[pl.ds(gi * 16, 16)] + p1_v[pl.ds(gi * 16, 16)] + 1.0
            i = lax.bitcast_convert_type(deg, jnp.int32)
            i = jnp.full((16,), 0x5F3759DF, jnp.int32) - (i >> 1)
            y = lax.bitcast_convert_type(i, jnp.float32)
            y = y * (1.5 - 0.5 * deg * y * y)
            y = y * (1.5 - 0.5 * deg * y * y)
            y = y * (1.5 - 0.5 * deg * y * y)
            for k in range(16):
                s = y[k]
                r = gi * 16 + k
                h_v[r, pl.ds(0, 16)] = h_v[r, pl.ds(0, 16)] * s
                h_v[r, pl.ds(16, 16)] = h_v[r, pl.ds(16, 16)] * s
                dis_v[r] = jnp.full((16,), s, jnp.float32)
            return carry

        lax.fori_loop(0, nrows // 16, grp, 0)
        pltpu.sync_copy(h_v.at[pl.ds(0, nrows)], g_out.at[pl.ds(base, nrows)])
        pltpu.sync_copy(dis_v.at[pl.ds(0, nrows)],
                        dis_out.at[pl.ds(base, nrows)])

    @pl.when(wid < NW - 1)
    def _():
        run(320)

    @pl.when(wid == NW - 1)
    def _():
        run(N - 320 * (NW - 1))


@functools.partial(
    pl.kernel,
    out_type=jax.ShapeDtypeStruct((NC, N, D_HID), jnp.float32),
    mesh=_mesh,
    compiler_params=_sc_params,
    scratch_types=[
        pltpu.VMEM((EPT,), jnp.int32),
        pltpu.VMEM((EPT,), jnp.int32),
        pltpu.VMEM((CH, D_HID), jnp.float32),
        pltpu.VMEM((CH, D_HID), jnp.float32),
        pltpu.VMEM((CH, D_HID), jnp.float32),
        pltpu.VMEM((CH, D_HID), jnp.float32),
        pltpu.VMEM((128, D_HID), jnp.float32),
        pltpu.VMEM_SHARED((NPAD, D_HID), jnp.float32),
        pltpu.SemaphoreType.DMA,
        pltpu.SemaphoreType.DMA,
        pltpu.SemaphoreType.DMA,
        pltpu.SemaphoreType.DMA,
        pltpu.SemaphoreType.DMA,
        pltpu.SemaphoreType.DMA,
        pltpu.SemaphoreType.DMA,
        pltpu.SemaphoreType.DMA,
    ],
)
def _sc_messages(ei_hbm, g_hbm, acc_out,
                 src_v, dst_v, msg0, msg1, msg2, msg3, zb, acc_sh,
                 sg0, sg1, sg2, sg3, ss0, ss1, ss2, ss3):
    cid = lax.axis_index("c")
    sid = lax.axis_index("s")
    wid = cid * NS + sid
    msg = [msg0, msg1, msg2, msg3]
    sg = [sg0, sg1, sg2, sg3]
    ss = [ss0, ss1, ss2, ss3]

    pltpu.sync_copy(ei_hbm.at[0, pl.ds(wid * EPT, EPT)], src_v)
    pltpu.sync_copy(ei_hbm.at[1, pl.ds(wid * EPT, EPT)], dst_v)
    for i in range(128 * D_HID // 16):
        zb[i // (D_HID // 16), pl.ds((i % (D_HID // 16)) * 16, 16)] = (
            jnp.zeros((16,), jnp.float32))
    for jj in range(RPT // 128):
        pltpu.sync_copy(zb, acc_sh.at[pl.ds(sid * RPT + jj * 128, 128)])
    plsc.subcore_barrier()

    def sidx(j):
        return src_v.at[pl.ds(j * CH, CH)]

    def didx(j):
        return dst_v.at[pl.ds(j * CH, CH)]

    for b in range(4):
        pltpu.async_copy(g_hbm.at[sidx(b)], msg[b], sg[b])

    def quad(q, carry):
        j0 = q * 4
        for b in range(4):
            j = j0 + b
            pltpu.make_async_copy(g_hbm.at[sidx(j)], msg[b], sg[b]).wait()
            pltpu.async_copy(msg[b], acc_sh.at[didx(j)], ss[b], add=True)
        for b in range(4):
            j = j0 + b
            pltpu.make_async_copy(msg[b], acc_sh.at[didx(j)], ss[b]).wait()
            pltpu.async_copy(g_hbm.at[sidx(j + 4)], msg[b], sg[b])
        return carry

    lax.fori_loop(0, NCH // 4 - 1, quad, 0)
    j0 = NCH - 6
    for b in range(4):
        j = j0 + b
        pltpu.make_async_copy(g_hbm.at[sidx(j)], msg[b], sg[b]).wait()
        pltpu.async_copy(msg[b], acc_sh.at[didx(j)], ss[b], add=True)
    for b in range(2):
        j = j0 + b
        pltpu.make_async_copy(msg[b], acc_sh.at[didx(j)], ss[b]).wait()
        pltpu.async_copy(g_hbm.at[sidx(j + 4)], msg[b], sg[b])
    for b in range(2):
        j = j0 + 4 + b
        pltpu.make_async_copy(g_hbm.at[sidx(j)], msg[b], sg[b]).wait()
        pltpu.async_copy(msg[b], acc_sh.at[didx(j)], ss[b], add=True)
        pltpu.make_async_copy(msg[b], acc_sh.at[didx(j)], ss[b]).wait()
    for b in range(2, 4):
        j = j0 + b
        pltpu.make_async_copy(msg[b], acc_sh.at[didx(j)], ss[b]).wait()
    tsi = src_v.at[pl.ds(NCH * CH, TL)]
    tdi = dst_v.at[pl.ds(NCH * CH, TL)]
    tmsg = msg2.at[pl.ds(0, TL)]
    pltpu.async_copy(g_hbm.at[tsi], tmsg, sg2)
    pltpu.make_async_copy(g_hbm.at[tsi], tmsg, sg2).wait()
    pltpu.async_copy(tmsg, acc_sh.at[tdi], ss2, add=True)
    pltpu.make_async_copy(tmsg, acc_sh.at[tdi], ss2).wait()

    plsc.subcore_barrier()
    pltpu.sync_copy(acc_sh.at[pl.ds(sid * 625, 625)],
                    acc_out.at[cid, pl.ds(sid * 625, 625)])


@functools.partial(
    pl.kernel,
    out_type=jax.ShapeDtypeStruct((N, D_HID), jnp.float32),
    mesh=_mesh,
    compiler_params=_sc_params,
    scratch_types=[
        pltpu.VMEM((320, D_HID), jnp.float32),
        pltpu.VMEM((320, D_HID), jnp.float32),
        pltpu.VMEM((320, D_HID), jnp.float32),
        pltpu.VMEM((320, 16), jnp.float32),
        pltpu.VMEM((D_HID,), jnp.float32),
    ],
)
def _sc_final(acc_hbm, dis_hbm, g_hbm, b_hbm, out_hbm,
              a0_v, a1_v, g_v, dis_v, b_v):
    cid = lax.axis_index("c")
    sid = lax.axis_index("s")
    wid = cid * NS + sid

    def run(nrows):
        base = wid * 320
        pltpu.sync_copy(acc_hbm.at[0, pl.ds(base, nrows)],
                        a0_v.at[pl.ds(0, nrows)])
        pltpu.sync_copy(acc_hbm.at[1, pl.ds(base, nrows)],
                        a1_v.at[pl.ds(0, nrows)])
        pltpu.sync_copy(g_hbm.at[pl.ds(base, nrows)],
                        g_v.at[pl.ds(0, nrows)])
        pltpu.sync_copy(dis_hbm.at[pl.ds(base, nrows)],
                        dis_v.at[pl.ds(0, nrows)])
        pltpu.sync_copy(b_hbm, b_v)
        bv0 = b_v[pl.ds(0, 16)]
        bv1 = b_v[pl.ds(16, 16)]

        def row(r4, carry):
            for k in range(4):
                r = r4 * 4 + k
                dv = dis_v[r]
                v0 = (a0_v[r, pl.ds(0, 16)] + a1_v[r, pl.ds(0, 16)]
                      + g_v[r, pl.ds(0, 16)]) * dv + bv0
                v1 = (a0_v[r, pl.ds(16, 16)] + a1_v[r, pl.ds(16, 16)]
                      + g_v[r, pl.ds(16, 16)]) * dv + bv1
                a0_v[r, pl.ds(0, 16)] = jnp.maximum(v0, 0.0)
                a0_v[r, pl.ds(16, 16)] = jnp.maximum(v1, 0.0)
            return carry

        lax.fori_loop(0, nrows // 4, row, 0)
        pltpu.sync_copy(a0_v.at[pl.ds(0, nrows)],
                        out_hbm.at[pl.ds(base, nrows)])

    @pl.when(wid < NW - 1)
    def _():
        run(320)

    @pl.when(wid == NW - 1)
    def _():
        run(N - 320 * (NW - 1))


def kernel(x, edge_index, ptr, W, b):
    del ptr
    ei = edge_index.astype(jnp.int32)

    blk = 2000
    h = pl.pallas_call(
        _tc_matmul_body,
        grid=(N // blk,),
        in_specs=[
            pl.BlockSpec((blk, D_IN), lambda i: (i, 0)),
            pl.BlockSpec((D_IN, D_HID), lambda i: (0, 0)),
        ],
        out_specs=pl.BlockSpec((blk, D_HID), lambda i: (i, 0)),
        out_shape=jax.ShapeDtypeStruct((N, D_HID), jnp.float32),
    )(x, W)

    deg_parts = _sc_degree(ei)
    g, dis16 = _sc_prescale(h, deg_parts)
    acc_parts = _sc_messages(ei, g)
    return _sc_final(acc_parts, dis16, g, b)
